# baseline (speedup 1.0000x reference)
import functools

import jax
import jax.numpy as jnp
from jax import lax
from jax.experimental import pallas as pl
from jax.experimental.pallas import tpu as pltpu
from jax.experimental.pallas import tpu_sc as plsc

_VROWS = 1000001
_CTX = 50
_DIM = 32
_BATCH = 4096

_NC = 2
_NS = 16
_NW = _NC * _NS

_RPW = _BATCH // _NW
_NGRP = _RPW // 16
_FEAT = _CTX * _DIM
_DEPTH = 4

_BLK = 128
_NBLK_FULL = _VROWS // _BLK
_PACK = 128 // _DIM
_YROWS = 250008
_BPW = _NBLK_FULL // _NW
_NEXTRA = _NBLK_FULL - _BPW * _NW


def _transpose_block(src_v, dst_v, nrows, clamp):
    lanes = lax.iota(jnp.int32, 16)
    for h in range(8):
        rvec = lanes + 16 * (h % 2)
        for j in range(nrows):
            cvec = jnp.full((16,), min(_PACK * j + h // 2, clamp), jnp.int32)
            dst_v[j, pl.ds(16 * h, 16)] = plsc.load_gather(src_v, [rvec, cvec])


def _relayout_body(tt_ref, tail_ref, y_ref, in0, in1, out0, out1,
                   si0, si1, so0, so1):
    ins = (in0, in1)
    outs = (out0, out1)
    sin = (si0, si1)
    sout = (so0, so1)
    wid = lax.axis_index("s") * _NC + lax.axis_index("c")

    def src_col(t):
        return pl.multiple_of((wid + _NW * t) * _BLK, _BLK)

    def fire_in(t, p):
        return pltpu.async_copy(
            tt_ref.at[:, pl.ds(src_col(t), _BLK)], ins[p], sin[p])

    def wait_in(t, p):
        pltpu.make_async_copy(
            tt_ref.at[:, pl.ds(src_col(t), _BLK)], ins[p], sin[p]).wait()

    def fire_out(t, p):
        return pltpu.async_copy(
            outs[p], y_ref.at[pl.ds((wid + _NW * t) * _DIM, _DIM)], sout[p])

    def wait_out(t, p):
        pltpu.make_async_copy(
            outs[p], y_ref.at[pl.ds((wid + _NW * t) * _DIM, _DIM)],
            sout[p]).wait()

    fire_in(0, 0)
    fire_in(1, 1)

    def loop_body(k, _):
        for p in range(2):
            t = 2 * k + p
            wait_in(t, p)

            @pl.when(t >= 2)
            def _():
                wait_out(t - 2, p)

            _transpose_block(ins[p], outs[p], _DIM, 127)
            fire_out(t, p)

            @pl.when(t + 2 < _BPW)
            def _():
                fire_in(t + 2, p)

        return 0

    lax.fori_loop(0, _BPW // 2, loop_body, 0)
    wait_out(_BPW - 2, 0)
    wait_out(_BPW - 1, 1)

    @pl.when(wid < _NEXTRA)
    def _():
        col = pl.multiple_of((_BPW * _NW + wid) * _BLK, _BLK)
        pltpu.async_copy(
            tt_ref.at[:, pl.ds(col, _BLK)], in0, si0).wait()
        _transpose_block(in0, out0, _DIM, 127)
        pltpu.async_copy(
            out0, y_ref.at[pl.ds((_BPW * _NW + wid) * _DIM, _DIM)],
            so0).wait()

    @pl.when(wid == _NEXTRA)
    def _():
        pltpu.async_copy(tail_ref, in0, si0).wait()
        _transpose_block(in0, out0, 24, 127)
        pltpu.async_copy(
            out0.at[pl.ds(0, 24)], y_ref.at[pl.ds(999936 // _PACK, 24)],
            so0).wait()


@jax.jit
def _relayout(tt, tail_arr):
    mesh = plsc.VectorSubcoreMesh(
        core_axis_name="c", subcore_axis_name="s",
        num_cores=_NC, num_subcores=_NS)
    f = functools.partial(
        pl.kernel,
        out_type=jax.ShapeDtypeStruct((_YROWS, 128), jnp.float32),
        mesh=mesh,
        compiler_params=pltpu.CompilerParams(
            needs_layout_passes=False, use_tc_tiling_on_sc=True),
        scratch_types=[
            pltpu.VMEM((_DIM, _BLK), jnp.float32),
            pltpu.VMEM((_DIM, _BLK), jnp.float32),
            pltpu.VMEM((_DIM, 128), jnp.float32),
            pltpu.VMEM((_DIM, 128), jnp.float32),
            pltpu.SemaphoreType.DMA,
            pltpu.SemaphoreType.DMA,
            pltpu.SemaphoreType.DMA,
            pltpu.SemaphoreType.DMA,
        ],
    )(_relayout_body)
    return f(tt, tail_arr)


def _sc_body(ids_ref, table_ref, ws_ref, b_ref, out_ref,
             idx_raw, idx_v, r0, r1, r2, r3, ws_v, b_v, out_v,
             s0, s1, s2, s3):
    bufs = (r0, r1, r2, r3)
    sems = (s0, s1, s2, s3)
    wid = lax.axis_index("s") * _NC + lax.axis_index("c")
    col0 = wid * _RPW

    pltpu.sync_copy(ids_ref.at[pl.ds(col0, _RPW)], idx_raw)
    pltpu.sync_copy(ws_ref, ws_v)
    pltpu.sync_copy(b_ref, b_v)
    b_vec = b_v[pl.ds(0, 16)]
    lanes = lax.iota(jnp.int32, 16)
    rgs = [lanes + 16 * g for g in range(_NGRP)]

    def t_body(c, _):
        cm = jnp.full((16,), c, jnp.int32)
        for g in range(_NGRP):
            idx_v[c, pl.ds(16 * g, 16)] = plsc.load_gather(
                idx_raw, [rgs[g], cm])
        return 0

    lax.fori_loop(0, _CTX, t_body, 0)

    def fire(c, p):
        return pltpu.async_copy(table_ref.at[idx_v.at[c]], bufs[p], sems[p])

    def wait(c, p):
        pltpu.make_async_copy(table_ref.at[idx_v.at[c]], bufs[p],
                              sems[p]).wait()

    def compute(c, p, accs):
        buf = bufs[p]
        accs = list(accs)
        for m in range(_DIM):
            wv = ws_v[c * _DIM + m, pl.ds(0, 16)]
            cm = jnp.full((16,), m, jnp.int32)
            for g in range(_NGRP):
                accs[g] = accs[g] + plsc.load_gather(buf, [rgs[g], cm]) * wv
        return tuple(accs)

    for p in range(_DEPTH):
        fire(p, p)

    def loop_body(k, accs):
        for p in range(_DEPTH):
            c = _DEPTH * k + p
            wait(c, p)
            accs = compute(c, p, accs)

            @pl.when(c + _DEPTH < _CTX)
            def _():
                fire(c + _DEPTH, p)

        return accs

    accs = lax.fori_loop(0, _CTX // _DEPTH, loop_body,
                         tuple(b_vec for _ in range(_NGRP)))

    for p in range(_CTX % _DEPTH):
        c = _CTX - (_CTX % _DEPTH) + p
        wait(c, p)
        accs = compute(c, p, accs)

    for g in range(_NGRP):
        out_v[pl.ds(16 * g, 16)] = accs[g]
    pltpu.sync_copy(out_v, out_ref.at[pl.ds(col0, _RPW)])


@jax.jit
def _logits_sc(ids, table_lin, w_splat, b16):
    mesh = plsc.VectorSubcoreMesh(
        core_axis_name="c", subcore_axis_name="s",
        num_cores=_NC, num_subcores=_NS)
    f = functools.partial(
        pl.kernel,
        out_type=jax.ShapeDtypeStruct((_BATCH,), jnp.float32),
        mesh=mesh,
        compiler_params=pltpu.CompilerParams(
            needs_layout_passes=False, use_tc_tiling_on_sc=False),
        scratch_types=[
            pltpu.VMEM((_RPW, _CTX), jnp.int32),
            pltpu.VMEM((_CTX, _RPW), jnp.int32),
            pltpu.VMEM((_RPW, _DIM), jnp.float32),
            pltpu.VMEM((_RPW, _DIM), jnp.float32),
            pltpu.VMEM((_RPW, _DIM), jnp.float32),
            pltpu.VMEM((_RPW, _DIM), jnp.float32),
            pltpu.VMEM((_FEAT, 16), jnp.float32),
            pltpu.VMEM((16,), jnp.float32),
            pltpu.VMEM((_RPW,), jnp.float32),
            pltpu.SemaphoreType.DMA,
            pltpu.SemaphoreType.DMA,
            pltpu.SemaphoreType.DMA,
            pltpu.SemaphoreType.DMA,
        ],
    )(_sc_body)
    return f(ids, table_lin, w_splat, b16)


def kernel(input_ids, table, W, b):
    ids = input_ids.astype(jnp.int32)
    w_splat = jnp.broadcast_to(
        W.astype(jnp.float32).reshape(_FEAT, 1), (_FEAT, 16))
    b16 = jnp.broadcast_to(b.astype(jnp.float32), (16,))
    tail_arr = jnp.pad(table[999936:].T, ((0, 0), (0, 128 - (_VROWS - 999936))))
    y = _relayout(table.T, tail_arr)
    table_lin = y.reshape(_YROWS * _PACK, _DIM)
    return _logits_sc(ids, table_lin, w_splat, b16)


# speedup vs baseline: 1.2227x; 1.2227x over previous
import functools

import jax
import jax.numpy as jnp
from jax import lax
from jax.experimental import pallas as pl
from jax.experimental.pallas import tpu as pltpu
from jax.experimental.pallas import tpu_sc as plsc

_VROWS = 1000001
_CTX = 50
_DIM = 32
_BATCH = 4096

_NC = 2
_NS = 16
_NW = _NC * _NS

_RPW = _BATCH // _NW
_G = 16
_NGRP = _RPW // _G
_STREAM = 100
_NSTREAM = _G * _CTX // _STREAM
_FEAT = _CTX * _DIM

_BLK = 128
_NBLK_FULL = _VROWS // _BLK
_PACK = 128 // _DIM
_YROWS = 250008
_BPW = _NBLK_FULL // _NW
_NEXTRA = _NBLK_FULL - _BPW * _NW
_DIN = 8
_DOUT = 4
_TAILK = (_BPW // _DIN) * _DIN


def _transpose_block(src_v, dst_v, nrows, clamp):
    lanes = lax.iota(jnp.int32, 16)
    rv = [lanes, lanes + 16]

    def j_body(j, _):
        for h in range(8):
            cvec = jnp.minimum(
                jnp.full((16,), h // 2, jnp.int32) + _PACK * j,
                jnp.full((16,), clamp, jnp.int32))
            dst_v[j, pl.ds(16 * h, 16)] = plsc.load_gather(
                src_v, [rv[h % 2], cvec])
        return 0

    lax.fori_loop(0, nrows, j_body, 0)


def _relayout_body(tt_ref, tail_ref, y_ref, *refs):
    ins = refs[:_DIN]
    outs = refs[_DIN:_DIN + _DOUT]
    sin = refs[_DIN + _DOUT:2 * _DIN + _DOUT]
    sout = refs[2 * _DIN + _DOUT:]
    wid = lax.axis_index("s") * _NC + lax.axis_index("c")

    def src_col(t):
        return pl.multiple_of((wid + _NW * t) * _BLK, _BLK)

    def dst_row(t):
        return pl.multiple_of((wid + _NW * t) * _DIM, _DIM)

    def fire_in(t, p):
        return pltpu.async_copy(
            tt_ref.at[:, pl.ds(src_col(t), _BLK)], ins[p], sin[p])

    def wait_in(t, p):
        pltpu.make_async_copy(
            tt_ref.at[:, pl.ds(src_col(t), _BLK)], ins[p], sin[p]).wait()

    def fire_out(t, q):
        return pltpu.async_copy(
            outs[q], y_ref.at[pl.ds(dst_row(t), _DIM)], sout[q])

    def wait_out(t, q):
        pltpu.make_async_copy(
            outs[q], y_ref.at[pl.ds(dst_row(t), _DIM)], sout[q]).wait()

    for p in range(_DIN):
        fire_in(p, p)

    def loop_body(k, _):
        for p in range(_DIN):
            t = _DIN * k + p
            q = p % _DOUT
            wait_in(t, p)

            @pl.when(t >= _DOUT)
            def _():
                wait_out(t - _DOUT, q)

            _transpose_block(ins[p], outs[q], _DIM, 127)
            fire_out(t, q)

            @pl.when(t + _DIN < _BPW)
            def _():
                fire_in(t + _DIN, p)

        return 0

    lax.fori_loop(0, _TAILK // _DIN, loop_body, 0)

    for t in range(_TAILK, _BPW):
        p = t % _DIN
        q = p % _DOUT
        wait_in(t, p)
        wait_out(t - _DOUT, q)
        _transpose_block(ins[p], outs[q], _DIM, 127)
        fire_out(t, q)
    for t in range(_BPW - _DOUT, _BPW):
        wait_out(t, t % _DIN % _DOUT)

    @pl.when(wid < _NEXTRA)
    def _():
        col = pl.multiple_of((_BPW * _NW + wid) * _BLK, _BLK)
        pltpu.async_copy(
            tt_ref.at[:, pl.ds(col, _BLK)], ins[0], sin[0]).wait()
        _transpose_block(ins[0], outs[0], _DIM, 127)
        pltpu.async_copy(
            outs[0], y_ref.at[pl.ds((_BPW * _NW + wid) * _DIM, _DIM)],
            sout[0]).wait()

    @pl.when(wid == _NEXTRA)
    def _():
        pltpu.async_copy(tail_ref, ins[0], sin[0]).wait()
        _transpose_block(ins[0], outs[0], 24, 127)
        pltpu.async_copy(
            outs[0].at[pl.ds(0, 24)], y_ref.at[pl.ds(999936 // _PACK, 24)],
            sout[0]).wait()


@jax.jit
def _relayout(tt, tail_arr):
    mesh = plsc.VectorSubcoreMesh(
        core_axis_name="c", subcore_axis_name="s",
        num_cores=_NC, num_subcores=_NS)
    f = functools.partial(
        pl.kernel,
        out_type=jax.ShapeDtypeStruct((_YROWS, 128), jnp.float32),
        mesh=mesh,
        compiler_params=pltpu.CompilerParams(
            needs_layout_passes=False, use_tc_tiling_on_sc=True),
        scratch_types=(
            [pltpu.VMEM((_DIM, _BLK), jnp.float32)] * _DIN
            + [pltpu.VMEM((_DIM, 128), jnp.float32)] * _DOUT
            + [pltpu.SemaphoreType.DMA] * (_DIN + _DOUT)
        ),
    )(_relayout_body)
    return f(tt, tail_arr)


def _sc_body(ids_ref, table_ref, ws_ref, b_ref, out_ref,
             idx_v, rows0, rows1, ws_v, b_v, out_v, sem0, sem1):
    rows = (rows0, rows1)
    sems = (sem0, sem1)
    wid = lax.axis_index("s") * _NC + lax.axis_index("c")

    pltpu.sync_copy(ids_ref.at[wid], idx_v)
    pltpu.sync_copy(ws_ref, ws_v)
    pltpu.sync_copy(b_ref, b_v)
    b_vec = b_v[pl.ds(0, 16)]
    lane_row = lax.iota(jnp.int32, 16) * _CTX
    cols = [jnp.full((16,), m, jnp.int32) for m in range(_DIM)]

    def fire(g, p):
        for j in range(_NSTREAM):
            pltpu.async_copy(
                table_ref.at[idx_v.at[g * _NSTREAM + j]],
                rows[p].at[pl.ds(j * _STREAM, _STREAM)],
                sems[p],
            )

    def wait(g, p):
        for j in range(_NSTREAM):
            pltpu.make_async_copy(
                table_ref.at[idx_v.at[g * _NSTREAM + j]],
                rows[p].at[pl.ds(j * _STREAM, _STREAM)],
                sems[p],
            ).wait()

    def compute(p):
        buf = rows[p]

        def c_body(c, acc):
            ridx = lane_row + c
            for m in range(_DIM):
                g16 = plsc.load_gather(buf, [ridx, cols[m]])
                w16 = ws_v[c * _DIM + m, pl.ds(0, 16)]
                acc = acc + g16 * w16
            return acc

        return lax.fori_loop(0, _CTX, c_body, b_vec)

    fire(0, 0)
    fire(1, 1)

    def group_body(k, _):
        for p in range(2):
            g = 2 * k + p
            wait(g, p)
            acc = compute(p)
            out_v[pl.ds(g * _G, _G)] = acc

            @pl.when(g + 2 < _NGRP)
            def _():
                fire(g + 2, p)

        return 0

    lax.fori_loop(0, _NGRP // 2, group_body, 0)

    pltpu.sync_copy(out_v, out_ref.at[pl.ds(wid * _RPW, _RPW)])


@jax.jit
def _logits_sc(ids, table_lin, w_splat, b16):
    mesh = plsc.VectorSubcoreMesh(
        core_axis_name="c", subcore_axis_name="s",
        num_cores=_NC, num_subcores=_NS)
    f = functools.partial(
        pl.kernel,
        out_type=jax.ShapeDtypeStruct((_BATCH,), jnp.float32),
        mesh=mesh,
        compiler_params=pltpu.CompilerParams(
            needs_layout_passes=False, use_tc_tiling_on_sc=False),
        scratch_types=[
            pltpu.VMEM((_RPW * _CTX // _STREAM, _STREAM), jnp.int32),
            pltpu.VMEM((_G * _CTX, _DIM), jnp.float32),
            pltpu.VMEM((_G * _CTX, _DIM), jnp.float32),
            pltpu.VMEM((_FEAT, 16), jnp.float32),
            pltpu.VMEM((16,), jnp.float32),
            pltpu.VMEM((_RPW,), jnp.float32),
            pltpu.SemaphoreType.DMA,
            pltpu.SemaphoreType.DMA,
        ],
    )(_sc_body)
    return f(ids, table_lin, w_splat, b16)


def kernel(input_ids, table, W, b):
    ids = input_ids.astype(jnp.int32).reshape(
        _NW, _RPW * _CTX // _STREAM, _STREAM)
    w_splat = jnp.broadcast_to(
        W.astype(jnp.float32).reshape(_FEAT, 1), (_FEAT, 16))
    b16 = jnp.broadcast_to(b.astype(jnp.float32), (16,))
    tail_arr = jnp.pad(
        table[999936:].T, ((0, 0), (0, 128 - (_VROWS - 999936))))
    y = _relayout(table.T, tail_arr)
    table_lin = y.reshape(_YROWS * _PACK, _DIM)
    return _logits_sc(ids, table_lin, w_splat, b16)


# speedup vs baseline: 1.8031x; 1.4748x over previous
import functools

import jax
import jax.numpy as jnp
from jax import lax
from jax.experimental import pallas as pl
from jax.experimental.pallas import tpu as pltpu
from jax.experimental.pallas import tpu_sc as plsc

_VROWS = 1000001
_CTX = 50
_DIM = 32
_BATCH = 4096

_NC = 2
_NS = 16
_NW = _NC * _NS

_RPW = _BATCH // _NW
_G = 16
_NGRP = _RPW // _G
_STREAM = 100
_NSTREAM = _G * _CTX // _STREAM
_FEAT = _CTX * _DIM

_BLK = 128
_NBLK_FULL = _VROWS // _BLK
_PACK = 128 // _DIM
_YROWS = 250008
_BPW = _NBLK_FULL // _NW
_NEXTRA = _NBLK_FULL - _BPW * _NW
_DIN = 8
_DOUT = 4
_TAILK = (_BPW // _DIN) * _DIN


def _transpose_block(src_v, dst_v, nrows, clamp):
    lanes = lax.iota(jnp.int32, 16)
    rv = [lanes, lanes + 16]

    @plsc.parallel_loop(0, nrows, unroll=4)
    def j_body(j):
        vals = []
        for h in range(8):
            cvec = jnp.minimum(
                jnp.full((16,), h // 2, jnp.int32) + _PACK * j,
                jnp.full((16,), clamp, jnp.int32))
            vals.append(plsc.load_gather(src_v, [rv[h % 2], cvec]))
        for h in range(8):
            dst_v[j, pl.ds(16 * h, 16)] = vals[h]


def _relayout_body(tt_ref, tail_ref, y_ref, *refs):
    ins = refs[:_DIN]
    outs = refs[_DIN:_DIN + _DOUT]
    sin = refs[_DIN + _DOUT:2 * _DIN + _DOUT]
    sout = refs[2 * _DIN + _DOUT:]
    wid = lax.axis_index("s") * _NC + lax.axis_index("c")

    def src_col(t):
        return pl.multiple_of((wid + _NW * t) * _BLK, _BLK)

    def dst_row(t):
        return pl.multiple_of((wid + _NW * t) * _DIM, _DIM)

    def fire_in(t, p):
        return pltpu.async_copy(
            tt_ref.at[:, pl.ds(src_col(t), _BLK)], ins[p], sin[p])

    def wait_in(t, p):
        pltpu.make_async_copy(
            tt_ref.at[:, pl.ds(src_col(t), _BLK)], ins[p], sin[p]).wait()

    def fire_out(t, q):
        return pltpu.async_copy(
            outs[q], y_ref.at[pl.ds(dst_row(t), _DIM)], sout[q])

    def wait_out(t, q):
        pltpu.make_async_copy(
            outs[q], y_ref.at[pl.ds(dst_row(t), _DIM)], sout[q]).wait()

    for p in range(_DIN):
        fire_in(p, p)

    def loop_body(k, _):
        for p in range(_DIN):
            t = _DIN * k + p
            q = p % _DOUT
            wait_in(t, p)

            @pl.when(t >= _DOUT)
            def _():
                wait_out(t - _DOUT, q)

            _transpose_block(ins[p], outs[q], _DIM, 127)
            fire_out(t, q)

            @pl.when(t + _DIN < _BPW)
            def _():
                fire_in(t + _DIN, p)

        return 0

    lax.fori_loop(0, _TAILK // _DIN, loop_body, 0)

    for t in range(_TAILK, _BPW):
        p = t % _DIN
        q = p % _DOUT
        wait_in(t, p)
        wait_out(t - _DOUT, q)
        _transpose_block(ins[p], outs[q], _DIM, 127)
        fire_out(t, q)
    for t in range(_BPW - _DOUT, _BPW):
        wait_out(t, t % _DIN % _DOUT)

    @pl.when(wid < _NEXTRA)
    def _():
        col = pl.multiple_of((_BPW * _NW + wid) * _BLK, _BLK)
        pltpu.async_copy(
            tt_ref.at[:, pl.ds(col, _BLK)], ins[0], sin[0]).wait()
        _transpose_block(ins[0], outs[0], _DIM, 127)
        pltpu.async_copy(
            outs[0], y_ref.at[pl.ds((_BPW * _NW + wid) * _DIM, _DIM)],
            sout[0]).wait()

    @pl.when(wid == _NEXTRA)
    def _():
        pltpu.async_copy(tail_ref, ins[0], sin[0]).wait()
        _transpose_block(ins[0], outs[0], 24, 127)
        pltpu.async_copy(
            outs[0].at[pl.ds(0, 24)], y_ref.at[pl.ds(999936 // _PACK, 24)],
            sout[0]).wait()


@jax.jit
def _relayout(tt, tail_arr):
    mesh = plsc.VectorSubcoreMesh(
        core_axis_name="c", subcore_axis_name="s",
        num_cores=_NC, num_subcores=_NS)
    f = functools.partial(
        pl.kernel,
        out_type=jax.ShapeDtypeStruct((_YROWS, 128), jnp.float32),
        mesh=mesh,
        compiler_params=pltpu.CompilerParams(
            needs_layout_passes=False, use_tc_tiling_on_sc=True),
        scratch_types=(
            [pltpu.VMEM((_DIM, _BLK), jnp.float32)] * _DIN
            + [pltpu.VMEM((_DIM, 128), jnp.float32)] * _DOUT
            + [pltpu.SemaphoreType.DMA] * (_DIN + _DOUT)
        ),
    )(_relayout_body)
    return f(tt, tail_arr)


def _sc_body(ids_ref, table_ref, ws_ref, b_ref, out_ref,
             idx_v, rows0, rows1, ws_v, b_v, out_v, sem0, sem1):
    rows = (rows0, rows1)
    sems = (sem0, sem1)
    wid = lax.axis_index("s") * _NC + lax.axis_index("c")

    pltpu.sync_copy(ids_ref.at[wid], idx_v)
    pltpu.sync_copy(ws_ref, ws_v)
    pltpu.sync_copy(b_ref, b_v)
    b_vec = b_v[pl.ds(0, 16)]
    lane_row = lax.iota(jnp.int32, 16) * _CTX
    cols = [jnp.full((16,), m, jnp.int32) for m in range(_DIM)]

    def fire(g, p):
        for j in range(_NSTREAM):
            pltpu.async_copy(
                table_ref.at[idx_v.at[g * _NSTREAM + j]],
                rows[p].at[pl.ds(j * _STREAM, _STREAM)],
                sems[p],
            )

    def wait(g, p):
        for j in range(_NSTREAM):
            pltpu.make_async_copy(
                table_ref.at[idx_v.at[g * _NSTREAM + j]],
                rows[p].at[pl.ds(j * _STREAM, _STREAM)],
                sems[p],
            ).wait()

    def compute(p):
        buf = rows[p]

        def c_body(c, acc):
            ridx = lane_row + c
            for m in range(_DIM):
                g16 = plsc.load_gather(buf, [ridx, cols[m]])
                w16 = ws_v[c * _DIM + m, pl.ds(0, 16)]
                acc = acc + g16 * w16
            return acc

        return lax.fori_loop(0, _CTX, c_body, b_vec)

    fire(0, 0)
    fire(1, 1)

    def group_body(k, _):
        for p in range(2):
            g = 2 * k + p
            wait(g, p)
            acc = compute(p)
            out_v[pl.ds(g * _G, _G)] = acc

            @pl.when(g + 2 < _NGRP)
            def _():
                fire(g + 2, p)

        return 0

    lax.fori_loop(0, _NGRP // 2, group_body, 0)

    pltpu.sync_copy(out_v, out_ref.at[pl.ds(wid * _RPW, _RPW)])


@jax.jit
def _logits_sc(ids, table_lin, w_splat, b16):
    mesh = plsc.VectorSubcoreMesh(
        core_axis_name="c", subcore_axis_name="s",
        num_cores=_NC, num_subcores=_NS)
    f = functools.partial(
        pl.kernel,
        out_type=jax.ShapeDtypeStruct((_BATCH,), jnp.float32),
        mesh=mesh,
        compiler_params=pltpu.CompilerParams(
            needs_layout_passes=False, use_tc_tiling_on_sc=False),
        scratch_types=[
            pltpu.VMEM((_RPW * _CTX // _STREAM, _STREAM), jnp.int32),
            pltpu.VMEM((_G * _CTX, _DIM), jnp.float32),
            pltpu.VMEM((_G * _CTX, _DIM), jnp.float32),
            pltpu.VMEM((_FEAT, 16), jnp.float32),
            pltpu.VMEM((16,), jnp.float32),
            pltpu.VMEM((_RPW,), jnp.float32),
            pltpu.SemaphoreType.DMA,
            pltpu.SemaphoreType.DMA,
        ],
    )(_sc_body)
    return f(ids, table_lin, w_splat, b16)


def kernel(input_ids, table, W, b):
    ids = input_ids.astype(jnp.int32).reshape(
        _NW, _RPW * _CTX // _STREAM, _STREAM)
    w_splat = jnp.broadcast_to(
        W.astype(jnp.float32).reshape(_FEAT, 1), (_FEAT, 16))
    b16 = jnp.broadcast_to(b.astype(jnp.float32), (16,))
    tail_arr = jnp.pad(
        table[999936:].T, ((0, 0), (0, 128 - (_VROWS - 999936))))
    y = _relayout(table.T, tail_arr)
    table_lin = y.reshape(_YROWS * _PACK, _DIM)
    return _logits_sc(ids, table_lin, w_splat, b16)


# speedup vs baseline: 1.9444x; 1.0783x over previous
import functools

import jax
import jax.numpy as jnp
from jax import lax
from jax.experimental import pallas as pl
from jax.experimental.pallas import tpu as pltpu
from jax.experimental.pallas import tpu_sc as plsc

_VROWS = 1000001
_CTX = 50
_DIM = 32
_BATCH = 4096

_NC = 2
_NS = 16
_NW = _NC * _NS

_RPW = _BATCH // _NW
_G = 16
_NGRP = _RPW // _G
_STREAM = 100
_NSTREAM = _G * _CTX // _STREAM
_FEAT = _CTX * _DIM

_BLK = 512
_NBLK_FULL = _VROWS // _BLK
_PACK = 128 // _DIM
_YROWS = 250008
_YPB = _BLK // _PACK
_BPW = _NBLK_FULL // _NW
_NEXTRA = _NBLK_FULL - _BPW * _NW
_DIN = 4
_DOUT = 2
_TAILK = (_BPW // _DIN) * _DIN


def _transpose_block(src_v, dst_v, nrows, clamp):
    lanes = lax.iota(jnp.int32, 16)
    rv = [lanes, lanes + 16]

    @plsc.parallel_loop(0, nrows, unroll=4)
    def j_body(j):
        vals = []
        for h in range(8):
            cvec = jnp.minimum(
                jnp.full((16,), h // 2, jnp.int32) + _PACK * j,
                jnp.full((16,), clamp, jnp.int32))
            vals.append(plsc.load_gather(src_v, [rv[h % 2], cvec]))
        for h in range(8):
            dst_v[j, pl.ds(16 * h, 16)] = vals[h]


def _relayout_body(tt_ref, tail_ref, y_ref, *refs):
    ins = refs[:_DIN]
    outs = refs[_DIN:_DIN + _DOUT]
    sin = refs[_DIN + _DOUT:2 * _DIN + _DOUT]
    sout = refs[2 * _DIN + _DOUT:]
    wid = lax.axis_index("s") * _NC + lax.axis_index("c")

    def src_col(t):
        return pl.multiple_of((wid + _NW * t) * _BLK, _BLK)

    def dst_row(t):
        return pl.multiple_of((wid + _NW * t) * _YPB, _YPB)

    def fire_in(t, p):
        return pltpu.async_copy(
            tt_ref.at[:, pl.ds(src_col(t), _BLK)], ins[p], sin[p])

    def wait_in(t, p):
        pltpu.make_async_copy(
            tt_ref.at[:, pl.ds(src_col(t), _BLK)], ins[p], sin[p]).wait()

    def fire_out(t, q):
        return pltpu.async_copy(
            outs[q], y_ref.at[pl.ds(dst_row(t), _YPB)], sout[q])

    def wait_out(t, q):
        pltpu.make_async_copy(
            outs[q], y_ref.at[pl.ds(dst_row(t), _YPB)], sout[q]).wait()

    for p in range(_DIN):
        fire_in(p, p)

    def loop_body(k, _):
        for p in range(_DIN):
            t = _DIN * k + p
            q = p % _DOUT
            wait_in(t, p)

            @pl.when(t >= _DOUT)
            def _():
                wait_out(t - _DOUT, q)

            _transpose_block(ins[p], outs[q], _YPB, _BLK - 1)
            fire_out(t, q)

            @pl.when(t + _DIN < _BPW)
            def _():
                fire_in(t + _DIN, p)

        return 0

    lax.fori_loop(0, _TAILK // _DIN, loop_body, 0)

    for t in range(_TAILK, _BPW):
        p = t % _DIN
        q = p % _DOUT
        wait_in(t, p)
        wait_out(t - _DOUT, q)
        _transpose_block(ins[p], outs[q], _YPB, _BLK - 1)
        fire_out(t, q)
    for t in range(_BPW - _DOUT, _BPW):
        wait_out(t, t % _DIN % _DOUT)

    @pl.when(wid < _NEXTRA)
    def _():
        col = pl.multiple_of((_BPW * _NW + wid) * _BLK, _BLK)
        pltpu.async_copy(
            tt_ref.at[:, pl.ds(col, _BLK)], ins[0], sin[0]).wait()
        _transpose_block(ins[0], outs[0], _YPB, _BLK - 1)
        pltpu.async_copy(
            outs[0], y_ref.at[pl.ds((_BPW * _NW + wid) * _YPB, _YPB)],
            sout[0]).wait()

    @pl.when(wid == _NEXTRA)
    def _():
        pltpu.async_copy(tail_ref, ins[0].at[:, pl.ds(0, 128)], sin[0]).wait()
        _transpose_block(ins[0], outs[0], 24, 127)
        pltpu.async_copy(
            outs[0].at[pl.ds(0, 24)], y_ref.at[pl.ds(999936 // _PACK, 24)],
            sout[0]).wait()


@jax.jit
def _relayout(tt, tail_arr):
    mesh = plsc.VectorSubcoreMesh(
        core_axis_name="c", subcore_axis_name="s",
        num_cores=_NC, num_subcores=_NS)
    f = functools.partial(
        pl.kernel,
        out_type=jax.ShapeDtypeStruct((_YROWS, 128), jnp.float32),
        mesh=mesh,
        compiler_params=pltpu.CompilerParams(
            needs_layout_passes=False, use_tc_tiling_on_sc=True),
        scratch_types=(
            [pltpu.VMEM((_DIM, _BLK), jnp.float32)] * _DIN
            + [pltpu.VMEM((_YPB, 128), jnp.float32)] * _DOUT
            + [pltpu.SemaphoreType.DMA] * (_DIN + _DOUT)
        ),
    )(_relayout_body)
    return f(tt, tail_arr)


def _sc_body(ids_ref, table_ref, ws_ref, b_ref, out_ref,
             idx_v, rows0, rows1, ws_v, b_v, out_v, sem0, sem1):
    rows = (rows0, rows1)
    sems = (sem0, sem1)
    wid = lax.axis_index("s") * _NC + lax.axis_index("c")

    pltpu.sync_copy(ids_ref.at[wid], idx_v)
    pltpu.sync_copy(ws_ref, ws_v)
    pltpu.sync_copy(b_ref, b_v)
    b_vec = b_v[pl.ds(0, 16)]
    lane_row = lax.iota(jnp.int32, 16) * _CTX
    cols = [jnp.full((16,), m, jnp.int32) for m in range(_DIM)]

    def fire(g, p):
        for j in range(_NSTREAM):
            pltpu.async_copy(
                table_ref.at[idx_v.at[g * _NSTREAM + j]],
                rows[p].at[pl.ds(j * _STREAM, _STREAM)],
                sems[p],
            )

    def wait(g, p):
        for j in range(_NSTREAM):
            pltpu.make_async_copy(
                table_ref.at[idx_v.at[g * _NSTREAM + j]],
                rows[p].at[pl.ds(j * _STREAM, _STREAM)],
                sems[p],
            ).wait()

    def compute(p):
        buf = rows[p]

        def c_body(c, acc):
            ridx = lane_row + c
            for m in range(_DIM):
                g16 = plsc.load_gather(buf, [ridx, cols[m]])
                w16 = ws_v[c * _DIM + m, pl.ds(0, 16)]
                acc = acc + g16 * w16
            return acc

        return lax.fori_loop(0, _CTX, c_body, b_vec)

    fire(0, 0)
    fire(1, 1)

    def group_body(k, _):
        for p in range(2):
            g = 2 * k + p
            wait(g, p)
            acc = compute(p)
            out_v[pl.ds(g * _G, _G)] = acc

            @pl.when(g + 2 < _NGRP)
            def _():
                fire(g + 2, p)

        return 0

    lax.fori_loop(0, _NGRP // 2, group_body, 0)

    pltpu.sync_copy(out_v, out_ref.at[pl.ds(wid * _RPW, _RPW)])


@jax.jit
def _logits_sc(ids, table_lin, w_splat, b16):
    mesh = plsc.VectorSubcoreMesh(
        core_axis_name="c", subcore_axis_name="s",
        num_cores=_NC, num_subcores=_NS)
    f = functools.partial(
        pl.kernel,
        out_type=jax.ShapeDtypeStruct((_BATCH,), jnp.float32),
        mesh=mesh,
        compiler_params=pltpu.CompilerParams(
            needs_layout_passes=False, use_tc_tiling_on_sc=False),
        scratch_types=[
            pltpu.VMEM((_RPW * _CTX // _STREAM, _STREAM), jnp.int32),
            pltpu.VMEM((_G * _CTX, _DIM), jnp.float32),
            pltpu.VMEM((_G * _CTX, _DIM), jnp.float32),
            pltpu.VMEM((_FEAT, 16), jnp.float32),
            pltpu.VMEM((16,), jnp.float32),
            pltpu.VMEM((_RPW,), jnp.float32),
            pltpu.SemaphoreType.DMA,
            pltpu.SemaphoreType.DMA,
        ],
    )(_sc_body)
    return f(ids, table_lin, w_splat, b16)


def kernel(input_ids, table, W, b):
    ids = input_ids.astype(jnp.int32).reshape(
        _NW, _RPW * _CTX // _STREAM, _STREAM)
    w_splat = jnp.broadcast_to(
        W.astype(jnp.float32).reshape(_FEAT, 1), (_FEAT, 16))
    b16 = jnp.broadcast_to(b.astype(jnp.float32), (16,))
    tail_arr = jnp.pad(
        table[999936:].T, ((0, 0), (0, 128 - (_VROWS - 999936))))
    y = _relayout(table.T, tail_arr)
    table_lin = y.reshape(_YROWS * _PACK, _DIM)
    return _logits_sc(ids, table_lin, w_splat, b16)
